# baseline (speedup 1.0000x reference)
import jax
import jax.numpy as jnp
from jax.experimental import pallas as pl
from jax.experimental.pallas import tpu as pltpu

NUM_SELECTS = 8
BT = 512


def _gate_body(x_ref, wt_ref, idx_ref, score_ref):
    x_bf = x_ref[...].astype(jnp.bfloat16)
    logits = jnp.dot(x_bf, wt_ref[...], preferred_element_type=jnp.float32)

    bt, e = logits.shape
    colv = jax.lax.broadcasted_iota(jnp.int32, (e, 1), 0).astype(jnp.float32)
    neg_inf = jnp.float32(-jnp.inf)

    work = logits
    vals = []
    idxs = []
    for _ in range(NUM_SELECTS):
        m = jnp.max(work, axis=1, keepdims=True)
        hit = work == m
        hit_f = jnp.where(hit, jnp.float32(1.0), jnp.float32(0.0))
        idx = jnp.dot(hit_f, colv, preferred_element_type=jnp.float32)
        vals.append(m)
        idxs.append(idx)
        work = jnp.where(hit, neg_inf, work)

    v = jnp.concatenate(vals, axis=1)
    i = jnp.concatenate(idxs, axis=1)
    ex = jnp.exp(v - v[:, 0:1])
    s = ex / jnp.sum(ex, axis=1, keepdims=True)
    idx_ref[...] = i.astype(jnp.int32)
    score_ref[...] = s


@jax.jit
def kernel(x, gate_weight):
    t, d = x.shape
    e = gate_weight.shape[0]
    wt = gate_weight.T.astype(jnp.bfloat16)
    grid = (t // BT,)
    idx, score = pl.pallas_call(
        _gate_body,
        grid=grid,
        in_specs=[
            pl.BlockSpec((BT, d), lambda i: (i, 0)),
            pl.BlockSpec((d, e), lambda i: (0, 0)),
        ],
        out_specs=[
            pl.BlockSpec((BT, NUM_SELECTS), lambda i: (i, 0)),
            pl.BlockSpec((BT, NUM_SELECTS), lambda i: (i, 0)),
        ],
        out_shape=[
            jax.ShapeDtypeStruct((t, NUM_SELECTS), jnp.int32),
            jax.ShapeDtypeStruct((t, NUM_SELECTS), jnp.float32),
        ],
        compiler_params=pltpu.CompilerParams(
            dimension_semantics=("arbitrary",),
        ),
    )(x, wt)
    return idx, score


# speedup vs baseline: 1.7050x; 1.7050x over previous
import jax
import jax.numpy as jnp
from jax.experimental import pallas as pl
from jax.experimental.pallas import tpu as pltpu

NUM_SELECTS = 8
BT = 512
CHUNK = 64


def _topk_chunk(logits, colf):
    e = logits.shape[1]
    neg_inf = jnp.float32(-jnp.inf)
    big = jnp.float32(e)
    work = logits
    vals = []
    idxs = []
    for _ in range(NUM_SELECTS):
        m = jnp.max(work, axis=1, keepdims=True)
        hit = work == m
        idx = jnp.min(jnp.where(hit, colf, big), axis=1, keepdims=True)
        vals.append(m)
        idxs.append(idx)
        work = jnp.where(hit, neg_inf, work)
    v = jnp.concatenate(vals, axis=1)
    i = jnp.concatenate(idxs, axis=1)
    ex = jnp.exp(v - v[:, 0:1])
    s = ex / jnp.sum(ex, axis=1, keepdims=True)
    return i, s


def _gate_body(x_ref, wt_ref, idx_ref, score_ref):
    x_bf = x_ref[...].astype(jnp.bfloat16)
    logits = jnp.dot(x_bf, wt_ref[...], preferred_element_type=jnp.float32)
    bt, e = logits.shape
    colf = jax.lax.broadcasted_iota(jnp.int32, (CHUNK, e), 1).astype(jnp.float32)
    for c in range(bt // CHUNK):
        sl = slice(c * CHUNK, (c + 1) * CHUNK)
        i, s = _topk_chunk(logits[sl, :], colf)
        idx_ref[sl, :] = i.astype(jnp.int32)
        score_ref[sl, :] = s


@jax.jit
def kernel(x, gate_weight):
    t, d = x.shape
    e = gate_weight.shape[0]
    wt = gate_weight.T.astype(jnp.bfloat16)
    grid = (t // BT,)
    idx, score = pl.pallas_call(
        _gate_body,
        grid=grid,
        in_specs=[
            pl.BlockSpec((BT, d), lambda i: (i, 0)),
            pl.BlockSpec((d, e), lambda i: (0, 0)),
        ],
        out_specs=[
            pl.BlockSpec((BT, NUM_SELECTS), lambda i: (i, 0)),
            pl.BlockSpec((BT, NUM_SELECTS), lambda i: (i, 0)),
        ],
        out_shape=[
            jax.ShapeDtypeStruct((t, NUM_SELECTS), jnp.int32),
            jax.ShapeDtypeStruct((t, NUM_SELECTS), jnp.float32),
        ],
        compiler_params=pltpu.CompilerParams(
            dimension_semantics=("arbitrary",),
        ),
    )(x, wt)
    return idx, score


# speedup vs baseline: 1.8154x; 1.0647x over previous
import jax
import jax.numpy as jnp
from jax.experimental import pallas as pl
from jax.experimental.pallas import tpu as pltpu

NUM_SELECTS = 8
BT = 1024
CHUNK = 64


def _topk_chunk(logits, colf):
    e = logits.shape[1]
    neg_inf = jnp.float32(-jnp.inf)
    big = jnp.float32(e)
    work = logits
    vals = []
    idxs = []
    for _ in range(NUM_SELECTS):
        m = jnp.max(work, axis=1, keepdims=True)
        hit = work == m
        idx = jnp.min(jnp.where(hit, colf, big), axis=1, keepdims=True)
        vals.append(m)
        idxs.append(idx)
        work = jnp.where(hit, neg_inf, work)
    v = jnp.concatenate(vals, axis=1)
    i = jnp.concatenate(idxs, axis=1)
    ex = jnp.exp(v - v[:, 0:1])
    s = ex / jnp.sum(ex, axis=1, keepdims=True)
    return i, s


def _gate_body(x_ref, wt_ref, idx_ref, score_ref):
    x_bf = x_ref[...].astype(jnp.bfloat16)
    logits = jnp.dot(x_bf, wt_ref[...], preferred_element_type=jnp.float32)
    bt, e = logits.shape
    colf = jax.lax.broadcasted_iota(jnp.int32, (CHUNK, e), 1).astype(jnp.float32)
    for c in range(bt // CHUNK):
        sl = slice(c * CHUNK, (c + 1) * CHUNK)
        i, s = _topk_chunk(logits[sl, :], colf)
        idx_ref[sl, :] = i.astype(jnp.int32)
        score_ref[sl, :] = s


@jax.jit
def kernel(x, gate_weight):
    t, d = x.shape
    e = gate_weight.shape[0]
    wt = gate_weight.T.astype(jnp.bfloat16)
    grid = (t // BT,)
    idx, score = pl.pallas_call(
        _gate_body,
        grid=grid,
        in_specs=[
            pl.BlockSpec((BT, d), lambda i: (i, 0)),
            pl.BlockSpec((d, e), lambda i: (0, 0)),
        ],
        out_specs=[
            pl.BlockSpec((BT, NUM_SELECTS), lambda i: (i, 0)),
            pl.BlockSpec((BT, NUM_SELECTS), lambda i: (i, 0)),
        ],
        out_shape=[
            jax.ShapeDtypeStruct((t, NUM_SELECTS), jnp.int32),
            jax.ShapeDtypeStruct((t, NUM_SELECTS), jnp.float32),
        ],
        compiler_params=pltpu.CompilerParams(
            dimension_semantics=("arbitrary",),
        ),
    )(x, wt)
    return idx, score
